# baseline (speedup 1.0000x reference)
import functools

import jax
import jax.numpy as jnp
from jax import lax
from jax.experimental import pallas as pl
from jax.experimental.pallas import tpu as pltpu
from jax.experimental.pallas import tpu_sc as plsc

B, N, K, C = 4, 8192, 8, 128
M = B * N
NC, NS, L = 2, 16, 16
NW = NC * NS
NPT = M // NW
CH = 32
NCHUNK = NPT // CH
PB = 2048


def _prep_body(ci_ref, gidx_ref, elr_ref):
    pid = pl.program_id(0)
    base = (pid * PB // N) * N
    ci = ci_ref[...]
    m = (ci != 0).astype(jnp.float32)
    num_sib = jnp.sum(m, axis=1, keepdims=True)
    is1 = num_sib == 1.0
    denom = jnp.where(is1, 1.0, num_sib - 1.0)
    kidx = lax.broadcasted_iota(jnp.int32, (PB, K), 1).astype(jnp.float32)
    er_full = jnp.where(is1, jnp.where(kidx == 0.0, 0.5, 0.0),
                        kidx * m / denom)
    elr_ref[...] = jnp.concatenate([m * (1.0 - er_full), m * er_full], axis=1)
    gidx_ref[...] = ci + base


def _prep(ci_flat):
    return pl.pallas_call(
        _prep_body,
        grid=(M // PB,),
        in_specs=[pl.BlockSpec((PB, K), lambda i: (i, 0))],
        out_specs=[pl.BlockSpec((PB, K), lambda i: (i, 0)),
                   pl.BlockSpec((PB, 2 * K), lambda i: (i, 0))],
        out_shape=[
            jax.ShapeDtypeStruct((M, K), jnp.int32),
            jax.ShapeDtypeStruct((M, 2 * K), jnp.float32),
        ],
    )(ci_flat)


_BCAST_DN = lax.GatherDimensionNumbers(
    offset_dims=(), collapsed_slice_dims=(0,), start_index_map=(0,))


def _lane_bcast(vec16, k):
    idx = jnp.full((16, 1), k, jnp.int32)
    return lax.gather(vec16, idx, _BCAST_DN, slice_sizes=(1,),
                      mode=lax.GatherScatterMode.PROMISE_IN_BOUNDS)


def _sc_body(table, gidx, elr, sl, sr,
             idx0, idx1, w0, w1, rows0, rows1,
             sl0, sl1, sr0, sr1, gsem0, gsem1, osem0, osem1):
    wid = lax.axis_index("s") * NC + lax.axis_index("c")
    nbase = wid * NPT
    idx = [idx0, idx1]
    wv = [w0, w1]
    rows = [rows0, rows1]
    stl = [sl0, sl1]
    str_ = [sr0, sr1]
    gsem = [gsem0, gsem1]
    osem = [osem0, osem1]

    def issue(q, b):
        col = nbase + q * CH
        pltpu.sync_copy(gidx.at[pl.ds(col * K, CH * K)], idx[b])
        pltpu.sync_copy(elr.at[pl.ds(col * 2 * K, CH * 2 * K)], wv[b])
        pltpu.async_copy(table.at[idx[b]], rows[b], gsem[b])

    issue(0, 0)

    def outer(qq, carry):
        for b in range(2):
            q = qq * 2 + b

            @pl.when(q + 1 < NCHUNK)
            def _():
                issue(q + 1, (b + 1) % 2)

            pltpu.make_async_copy(table.at[idx[b]], rows[b], gsem[b]).wait()

            rows_v, elr_v, stagl, stagr = rows[b], wv[b], stl[b], str_[b]

            def node_body(i, c3):
                w16 = elr_v[pl.ds(i * (2 * K), 16)]
                accl = [None] * (C // 16)
                accr = [None] * (C // 16)
                for k in range(K):
                    bl = _lane_bcast(w16, k)
                    br = _lane_bcast(w16, K + k)
                    row = i * K + k
                    for v in range(C // 16):
                        r = rows_v[row, pl.ds(v * 16, 16)]
                        if k == 0:
                            accl[v] = bl * r
                            accr[v] = br * r
                        else:
                            accl[v] = accl[v] + bl * r
                            accr[v] = accr[v] + br * r
                for v in range(C // 16):
                    stagl[i, pl.ds(v * 16, 16)] = accl[v]
                    stagr[i, pl.ds(v * 16, 16)] = accr[v]
                return c3

            lax.fori_loop(0, CH, node_body, 0, unroll=8)

            col = nbase + q * CH
            pltpu.sync_copy(stagl, sl.at[pl.ds(col, CH), :])
            pltpu.sync_copy(stagr, sr.at[pl.ds(col, CH), :])
        return carry

    lax.fori_loop(0, NCHUNK // 2, outer, 0)


def _sc_gather(table, gidx_f, elr_f):
    mesh = plsc.VectorSubcoreMesh(core_axis_name="c", subcore_axis_name="s",
                                  num_cores=NC, num_subcores=NS)
    f = pl.kernel(
        _sc_body,
        out_type=[
            jax.ShapeDtypeStruct((M, C), jnp.float32),
            jax.ShapeDtypeStruct((M, C), jnp.float32),
        ],
        mesh=mesh,
        scratch_types=(
            [pltpu.VMEM((CH * K,), jnp.int32)] * 2
            + [pltpu.VMEM((CH * 2 * K,), jnp.float32)] * 2
            + [pltpu.VMEM((CH * K, C), jnp.float32)] * 2
            + [pltpu.VMEM((CH, C), jnp.float32)] * 4
            + [pltpu.SemaphoreType.DMA] * 4
        ),
        compiler_params=pltpu.CompilerParams(needs_layout_passes=False,
                                             disable_bounds_checks=True),
    )
    return f(table, gidx_f, elr_f)


def _conv0_body(acc_ref, sl_ref, sr_ref, wt_ref, wl_ref, wr_ref, b_ref,
                node_ref, acc1_ref):
    x = acc_ref[...]
    r = jnp.dot(x, wt_ref[...], preferred_element_type=jnp.float32)
    r += jnp.dot(sl_ref[...], wl_ref[...], preferred_element_type=jnp.float32)
    r += jnp.dot(sr_ref[...], wr_ref[...], preferred_element_type=jnp.float32)
    r += b_ref[...]
    node = jnp.where(r > 0, r, 0.01 * r)
    node_ref[...] = node
    acc1_ref[...] = x + node


def _conv1_body(acc_ref, sl_ref, sr_ref, wt_ref, wl_ref, wr_ref, b_ref,
                node_ref):
    x = acc_ref[...]
    r = jnp.dot(x, wt_ref[...], preferred_element_type=jnp.float32)
    r += jnp.dot(sl_ref[...], wl_ref[...], preferred_element_type=jnp.float32)
    r += jnp.dot(sr_ref[...], wr_ref[...], preferred_element_type=jnp.float32)
    r += b_ref[...]
    node_ref[...] = jnp.where(r > 0, r, 0.01 * r)


def _conv(acc, sl, sr, wt, wl, wr, b, want_acc):
    body = _conv0_body if want_acc else _conv1_body
    nout = 2 if want_acc else 1
    full = lambda i: (0, 0)
    out = pl.pallas_call(
        body,
        grid=(M // PB,),
        in_specs=[
            pl.BlockSpec((PB, C), lambda i: (i, 0)),
            pl.BlockSpec((PB, C), lambda i: (i, 0)),
            pl.BlockSpec((PB, C), lambda i: (i, 0)),
            pl.BlockSpec((C, C), full),
            pl.BlockSpec((C, C), full),
            pl.BlockSpec((C, C), full),
            pl.BlockSpec((1, C), full),
        ],
        out_specs=[pl.BlockSpec((PB, C), lambda i: (i, 0))] * nout,
        out_shape=[jax.ShapeDtypeStruct((M, C), jnp.float32)] * nout,
    )(acc, sl, sr, wt, wl, wr, b.reshape(1, C))
    return out if want_acc else out[0]


def _attn_body(x_ref, wq_ref, bq_ref, wk_ref, wv_ref, bv_ref, gate_ref,
               out_ref):
    x = x_ref[0]
    root = x[0:1, :]
    q = lax.dot_general(root, wq_ref[...], (((1,), (1,)), ((), ())),
                        preferred_element_type=jnp.float32) + bq_ref[...]
    kq = jnp.dot(q, wk_ref[...], preferred_element_type=jnp.float32)
    logits = lax.dot_general(x, kq, (((1,), (1,)), ((), ())),
                             preferred_element_type=jnp.float32)
    row = lax.broadcasted_iota(jnp.int32, (N, 1), 0)
    logits = jnp.where(row == 0, -1e30, logits)
    mx = jnp.max(logits)
    s = jnp.exp(logits - mx)
    s = jnp.where(row == 0, 0.0, s)
    z = jnp.sum(s)
    t = lax.dot_general(s, x, (((0,), (0,)), ((), ())),
                        preferred_element_type=jnp.float32)
    agg = lax.dot_general(t, wv_ref[...], (((1,), (1,)), ((), ())),
                          preferred_element_type=jnp.float32) / z
    agg = agg + bv_ref[...]
    g = jax.nn.sigmoid(gate_ref[0])
    out_ref[...] = (g * root + (1.0 - g) * agg).reshape(1, 1, C)


def _attention(node, Wq, bq, Wk, Wv, bv, gate):
    full = lambda i: (0, 0)
    return pl.pallas_call(
        _attn_body,
        grid=(B,),
        in_specs=[
            pl.BlockSpec((1, N, C), lambda i: (i, 0, 0)),
            pl.BlockSpec((C, C), full),
            pl.BlockSpec((1, C), full),
            pl.BlockSpec((C, C), full),
            pl.BlockSpec((C, C), full),
            pl.BlockSpec((1, C), full),
            pl.BlockSpec(memory_space=pltpu.SMEM),
        ],
        out_specs=pl.BlockSpec((1, 1, C), lambda i: (i, 0, 0)),
        out_shape=jax.ShapeDtypeStruct((B, 1, C), jnp.float32),
    )(node.reshape(B, N, C), Wq, bq.reshape(1, C), Wk, Wv,
      bv.reshape(1, C), gate).reshape(B, C)


def kernel(parent_node_embedding, children_index, w_t0, w_l0, w_r0, b0,
           w_t1, w_l1, w_r1, b1, Wq, bq, Wk, bk, Wv, bv, gate):
    parent = parent_node_embedding.reshape(M, C)
    ci = children_index.reshape(M, K)

    gidx, elr = _prep(ci)
    gidx_f = gidx.reshape(M * K)
    elr_f = elr.reshape(M * 2 * K)

    sl, sr = _sc_gather(parent, gidx_f, elr_f)
    node0, acc1 = _conv(parent, sl, sr, w_t0, w_l0, w_r0, b0, True)

    sl, sr = _sc_gather(node0, gidx_f, elr_f)
    node1 = _conv(acc1, sl, sr, w_t1, w_l1, w_r1, b1, False)

    return _attention(node1, Wq, bq, Wk, Wv, bv, gate)


# speedup vs baseline: 1.0764x; 1.0764x over previous
import functools

import jax
import jax.numpy as jnp
from jax import lax
from jax.experimental import pallas as pl
from jax.experimental.pallas import tpu as pltpu
from jax.experimental.pallas import tpu_sc as plsc

B, N, K, C = 4, 8192, 8, 128
M = B * N
NC, NS, L = 2, 16, 16
NW = NC * NS
NPT = M // NW
CH = 32
NCHUNK = NPT // CH
PB = 2048


def _prep_body(ci_ref, gidx_ref, elr_ref):
    pid = pl.program_id(0)
    base = (pid * PB // N) * N
    ci = ci_ref[...]
    m = (ci != 0).astype(jnp.float32)
    num_sib = jnp.sum(m, axis=1, keepdims=True)
    is1 = num_sib == 1.0
    denom = jnp.where(is1, 1.0, num_sib - 1.0)
    kidx = lax.broadcasted_iota(jnp.int32, (PB, K), 1).astype(jnp.float32)
    er_full = jnp.where(is1, jnp.where(kidx == 0.0, 0.5, 0.0),
                        kidx * m / denom)
    elr_ref[...] = jnp.concatenate([m * (1.0 - er_full), m * er_full], axis=1)
    gidx_ref[...] = ci + base


def _prep(ci_flat):
    return pl.pallas_call(
        _prep_body,
        grid=(M // PB,),
        in_specs=[pl.BlockSpec((PB, K), lambda i: (i, 0))],
        out_specs=[pl.BlockSpec((PB, K), lambda i: (i, 0)),
                   pl.BlockSpec((PB, 2 * K), lambda i: (i, 0))],
        out_shape=[
            jax.ShapeDtypeStruct((M, K), jnp.int32),
            jax.ShapeDtypeStruct((M, 2 * K), jnp.float32),
        ],
    )(ci_flat)


_BCAST_DN = lax.GatherDimensionNumbers(
    offset_dims=(), collapsed_slice_dims=(0,), start_index_map=(0,))


def _lane_bcast(vec16, k):
    idx = jnp.full((16, 1), k, jnp.int32)
    return lax.gather(vec16, idx, _BCAST_DN, slice_sizes=(1,),
                      mode=lax.GatherScatterMode.PROMISE_IN_BOUNDS)


def _sc_body(table, gidx, elr, sl, sr,
             idx0, idx1, w0, w1, rows0, rows1,
             sl0, sl1, sr0, sr1, gsem0, gsem1, osem0, osem1):
    wid = lax.axis_index("s") * NC + lax.axis_index("c")
    nbase = wid * NPT
    idx = [idx0, idx1]
    wv = [w0, w1]
    rows = [rows0, rows1]
    stl = [sl0, sl1]
    str_ = [sr0, sr1]
    gsem = [gsem0, gsem1]
    osem = [osem0, osem1]

    def issue(q, b):
        col = nbase + q * CH
        pltpu.sync_copy(gidx.at[pl.ds(col * K, CH * K)], idx[b])
        pltpu.sync_copy(elr.at[pl.ds(col * 2 * K, CH * 2 * K)], wv[b])
        pltpu.async_copy(table.at[idx[b]], rows[b], gsem[b])

    issue(0, 0)

    def outer(qq, carry):
        for b in range(2):
            q = qq * 2 + b

            @pl.when(q + 1 < NCHUNK)
            def _():
                issue(q + 1, (b + 1) % 2)

            pltpu.make_async_copy(table.at[idx[b]], rows[b], gsem[b]).wait()

            rows_v, elr_v, stagl, stagr = rows[b], wv[b], stl[b], str_[b]

            def node_body(i, c3):
                w16 = elr_v[pl.ds(i * (2 * K), 16)]
                accl = [None] * (C // 16)
                accr = [None] * (C // 16)
                for k in range(K):
                    bl = _lane_bcast(w16, k)
                    br = _lane_bcast(w16, K + k)
                    row = i * K + k
                    for v in range(C // 16):
                        r = rows_v[row, pl.ds(v * 16, 16)]
                        if k == 0:
                            accl[v] = bl * r
                            accr[v] = br * r
                        else:
                            accl[v] = accl[v] + bl * r
                            accr[v] = accr[v] + br * r
                for v in range(C // 16):
                    stagl[i, pl.ds(v * 16, 16)] = accl[v]
                    stagr[i, pl.ds(v * 16, 16)] = accr[v]
                return c3

            lax.fori_loop(0, CH, node_body, 0, unroll=4)

            col = nbase + q * CH
            pltpu.sync_copy(stagl, sl.at[pl.ds(col, CH), :])
            pltpu.sync_copy(stagr, sr.at[pl.ds(col, CH), :])
        return carry

    lax.fori_loop(0, NCHUNK // 2, outer, 0)


def _sc_gather(table, gidx_f, elr_f):
    mesh = plsc.VectorSubcoreMesh(core_axis_name="c", subcore_axis_name="s",
                                  num_cores=NC, num_subcores=NS)
    f = pl.kernel(
        _sc_body,
        out_type=[
            jax.ShapeDtypeStruct((M, C), jnp.float32),
            jax.ShapeDtypeStruct((M, C), jnp.float32),
        ],
        mesh=mesh,
        scratch_types=(
            [pltpu.VMEM((CH * K,), jnp.int32)] * 2
            + [pltpu.VMEM((CH * 2 * K,), jnp.float32)] * 2
            + [pltpu.VMEM((CH * K, C), jnp.float32)] * 2
            + [pltpu.VMEM((CH, C), jnp.float32)] * 4
            + [pltpu.SemaphoreType.DMA] * 4
        ),
        compiler_params=pltpu.CompilerParams(needs_layout_passes=False,
                                             disable_bounds_checks=True),
    )
    return f(table, gidx_f, elr_f)


def _conv0_body(acc_ref, sl_ref, sr_ref, wt_ref, wl_ref, wr_ref, b_ref,
                node_ref, acc1_ref):
    x = acc_ref[...]
    r = jnp.dot(x, wt_ref[...], preferred_element_type=jnp.float32)
    r += jnp.dot(sl_ref[...], wl_ref[...], preferred_element_type=jnp.float32)
    r += jnp.dot(sr_ref[...], wr_ref[...], preferred_element_type=jnp.float32)
    r += b_ref[...]
    node = jnp.where(r > 0, r, 0.01 * r)
    node_ref[...] = node
    acc1_ref[...] = x + node


def _conv1_body(acc_ref, sl_ref, sr_ref, wt_ref, wl_ref, wr_ref, b_ref,
                node_ref):
    x = acc_ref[...]
    r = jnp.dot(x, wt_ref[...], preferred_element_type=jnp.float32)
    r += jnp.dot(sl_ref[...], wl_ref[...], preferred_element_type=jnp.float32)
    r += jnp.dot(sr_ref[...], wr_ref[...], preferred_element_type=jnp.float32)
    r += b_ref[...]
    node_ref[...] = jnp.where(r > 0, r, 0.01 * r)


def _conv(acc, sl, sr, wt, wl, wr, b, want_acc):
    body = _conv0_body if want_acc else _conv1_body
    nout = 2 if want_acc else 1
    full = lambda i: (0, 0)
    out = pl.pallas_call(
        body,
        grid=(M // PB,),
        in_specs=[
            pl.BlockSpec((PB, C), lambda i: (i, 0)),
            pl.BlockSpec((PB, C), lambda i: (i, 0)),
            pl.BlockSpec((PB, C), lambda i: (i, 0)),
            pl.BlockSpec((C, C), full),
            pl.BlockSpec((C, C), full),
            pl.BlockSpec((C, C), full),
            pl.BlockSpec((1, C), full),
        ],
        out_specs=[pl.BlockSpec((PB, C), lambda i: (i, 0))] * nout,
        out_shape=[jax.ShapeDtypeStruct((M, C), jnp.float32)] * nout,
    )(acc, sl, sr, wt, wl, wr, b.reshape(1, C))
    return out if want_acc else out[0]


def _attn_body(x_ref, wq_ref, bq_ref, wk_ref, wv_ref, bv_ref, gate_ref,
               out_ref):
    x = x_ref[0]
    root = x[0:1, :]
    q = lax.dot_general(root, wq_ref[...], (((1,), (1,)), ((), ())),
                        preferred_element_type=jnp.float32) + bq_ref[...]
    kq = jnp.dot(q, wk_ref[...], preferred_element_type=jnp.float32)
    logits = lax.dot_general(x, kq, (((1,), (1,)), ((), ())),
                             preferred_element_type=jnp.float32)
    row = lax.broadcasted_iota(jnp.int32, (N, 1), 0)
    logits = jnp.where(row == 0, -1e30, logits)
    mx = jnp.max(logits)
    s = jnp.exp(logits - mx)
    s = jnp.where(row == 0, 0.0, s)
    z = jnp.sum(s)
    t = lax.dot_general(s, x, (((0,), (0,)), ((), ())),
                        preferred_element_type=jnp.float32)
    agg = lax.dot_general(t, wv_ref[...], (((1,), (1,)), ((), ())),
                          preferred_element_type=jnp.float32) / z
    agg = agg + bv_ref[...]
    g = jax.nn.sigmoid(gate_ref[0])
    out_ref[...] = (g * root + (1.0 - g) * agg).reshape(1, 1, C)


def _attention(node, Wq, bq, Wk, Wv, bv, gate):
    full = lambda i: (0, 0)
    return pl.pallas_call(
        _attn_body,
        grid=(B,),
        in_specs=[
            pl.BlockSpec((1, N, C), lambda i: (i, 0, 0)),
            pl.BlockSpec((C, C), full),
            pl.BlockSpec((1, C), full),
            pl.BlockSpec((C, C), full),
            pl.BlockSpec((C, C), full),
            pl.BlockSpec((1, C), full),
            pl.BlockSpec(memory_space=pltpu.SMEM),
        ],
        out_specs=pl.BlockSpec((1, 1, C), lambda i: (i, 0, 0)),
        out_shape=jax.ShapeDtypeStruct((B, 1, C), jnp.float32),
    )(node.reshape(B, N, C), Wq, bq.reshape(1, C), Wk, Wv,
      bv.reshape(1, C), gate).reshape(B, C)


def kernel(parent_node_embedding, children_index, w_t0, w_l0, w_r0, b0,
           w_t1, w_l1, w_r1, b1, Wq, bq, Wk, bk, Wv, bv, gate):
    parent = parent_node_embedding.reshape(M, C)
    ci = children_index.reshape(M, K)

    gidx, elr = _prep(ci)
    gidx_f = gidx.reshape(M * K)
    elr_f = elr.reshape(M * 2 * K)

    sl, sr = _sc_gather(parent, gidx_f, elr_f)
    node0, acc1 = _conv(parent, sl, sr, w_t0, w_l0, w_r0, b0, True)

    sl, sr = _sc_gather(node0, gidx_f, elr_f)
    node1 = _conv(acc1, sl, sr, w_t1, w_l1, w_r1, b1, False)

    return _attention(node1, Wq, bq, Wk, Wv, bv, gate)


# speedup vs baseline: 1.1313x; 1.0509x over previous
import functools

import jax
import jax.numpy as jnp
from jax import lax
from jax.experimental import pallas as pl
from jax.experimental.pallas import tpu as pltpu
from jax.experimental.pallas import tpu_sc as plsc

B, N, K, C = 4, 8192, 8, 128
M = B * N
NC, NS, L = 2, 16, 16
NW = NC * NS
NPT = M // NW
CH = 32
NCHUNK = NPT // CH
PB = 2048


def _prep_body(ci_ref, gidx_ref, elr_ref):
    pid = pl.program_id(0)
    base = (pid * PB // N) * N
    ci = ci_ref[...]
    m = (ci != 0).astype(jnp.float32)
    num_sib = jnp.sum(m, axis=1, keepdims=True)
    is1 = num_sib == 1.0
    denom = jnp.where(is1, 1.0, num_sib - 1.0)
    kidx = lax.broadcasted_iota(jnp.int32, (PB, K), 1).astype(jnp.float32)
    er_full = jnp.where(is1, jnp.where(kidx == 0.0, 0.5, 0.0),
                        kidx * m / denom)
    elr_ref[...] = jnp.concatenate([m * (1.0 - er_full), m * er_full], axis=1)
    gidx_ref[...] = ci + base


def _prep(ci_flat):
    return pl.pallas_call(
        _prep_body,
        grid=(M // PB,),
        in_specs=[pl.BlockSpec((PB, K), lambda i: (i, 0))],
        out_specs=[pl.BlockSpec((PB, K), lambda i: (i, 0)),
                   pl.BlockSpec((PB, 2 * K), lambda i: (i, 0))],
        out_shape=[
            jax.ShapeDtypeStruct((M, K), jnp.int32),
            jax.ShapeDtypeStruct((M, 2 * K), jnp.float32),
        ],
    )(ci_flat)


_BCAST_DN = lax.GatherDimensionNumbers(
    offset_dims=(), collapsed_slice_dims=(0,), start_index_map=(0,))


def _lane_bcast(vec16, k):
    idx = jnp.full((16, 1), k, jnp.int32)
    return lax.gather(vec16, idx, _BCAST_DN, slice_sizes=(1,),
                      mode=lax.GatherScatterMode.PROMISE_IN_BOUNDS)


def _sc_body(table, gidx, elr, sl, sr,
             idx0, idx1, w0, w1, rows0, rows1,
             sl0, sl1, sr0, sr1, gsem0, gsem1, osem0, osem1):
    wid = lax.axis_index("s") * NC + lax.axis_index("c")
    nbase = wid * NPT
    idx = [idx0, idx1]
    wv = [w0, w1]
    rows = [rows0, rows1]
    stl = [sl0, sl1]
    str_ = [sr0, sr1]
    gsem = [gsem0, gsem1]
    osem = [osem0, osem1]

    def issue(q, b):
        col = nbase + q * CH
        pltpu.sync_copy(gidx.at[pl.ds(col * K, CH * K)], idx[b])
        pltpu.sync_copy(elr.at[pl.ds(col * 2 * K, CH * 2 * K)], wv[b])
        pltpu.async_copy(table.at[idx[b]], rows[b], gsem[b])

    issue(0, 0)

    def outer(qq, carry):
        for b in range(2):
            q = qq * 2 + b

            @pl.when(q + 1 < NCHUNK)
            def _():
                issue(q + 1, (b + 1) % 2)

            pltpu.make_async_copy(table.at[idx[b]], rows[b], gsem[b]).wait()

            rows_v, elr_v, stagl, stagr = rows[b], wv[b], stl[b], str_[b]

            def node_body(i, c3):
                w16 = elr_v[pl.ds(i * (2 * K), 16)]
                accl = [None] * (C // 16)
                accr = [None] * (C // 16)
                for k in range(K):
                    bl = _lane_bcast(w16, k)
                    br = _lane_bcast(w16, K + k)
                    row = i * K + k
                    for v in range(C // 16):
                        r = rows_v[row, pl.ds(v * 16, 16)]
                        if k == 0:
                            accl[v] = bl * r
                            accr[v] = br * r
                        else:
                            accl[v] = accl[v] + bl * r
                            accr[v] = accr[v] + br * r
                for v in range(C // 16):
                    stagl[i, pl.ds(v * 16, 16)] = accl[v]
                    stagr[i, pl.ds(v * 16, 16)] = accr[v]
                return c3

            lax.fori_loop(0, CH, node_body, 0, unroll=4)

            col = nbase + q * CH
            pltpu.sync_copy(stagl, sl.at[pl.ds(col, CH), :])
            pltpu.sync_copy(stagr, sr.at[pl.ds(col, CH), :])
        return carry

    lax.fori_loop(0, NCHUNK // 2, outer, 0)


def _sc_gather(table, gidx_f, elr_f):
    mesh = plsc.VectorSubcoreMesh(core_axis_name="c", subcore_axis_name="s",
                                  num_cores=NC, num_subcores=NS)
    f = pl.kernel(
        _sc_body,
        out_type=[
            jax.ShapeDtypeStruct((M, C), jnp.float32),
            jax.ShapeDtypeStruct((M, C), jnp.float32),
        ],
        mesh=mesh,
        scratch_types=(
            [pltpu.VMEM((CH * K,), jnp.int32)] * 2
            + [pltpu.VMEM((CH * 2 * K,), jnp.float32)] * 2
            + [pltpu.VMEM((CH * K, C), jnp.float32)] * 2
            + [pltpu.VMEM((CH, C), jnp.float32)] * 4
            + [pltpu.SemaphoreType.DMA] * 4
        ),
        compiler_params=pltpu.CompilerParams(needs_layout_passes=False,
                                             disable_bounds_checks=True),
    )
    return f(table, gidx_f, elr_f)


def _conv0_body(acc_ref, sl_ref, sr_ref, wt_ref, wl_ref, wr_ref, b_ref,
                node_ref):
    x = acc_ref[...]
    r = jnp.dot(x, wt_ref[...], preferred_element_type=jnp.float32)
    r += jnp.dot(sl_ref[...], wl_ref[...], preferred_element_type=jnp.float32)
    r += jnp.dot(sr_ref[...], wr_ref[...], preferred_element_type=jnp.float32)
    r += b_ref[...]
    node_ref[...] = jnp.where(r > 0, r, 0.01 * r)


def _conv(acc, sl, sr, wt, wl, wr, b):
    full = lambda i: (0, 0)
    return pl.pallas_call(
        _conv0_body,
        grid=(M // PB,),
        in_specs=[
            pl.BlockSpec((PB, C), lambda i: (i, 0)),
            pl.BlockSpec((PB, C), lambda i: (i, 0)),
            pl.BlockSpec((PB, C), lambda i: (i, 0)),
            pl.BlockSpec((C, C), full),
            pl.BlockSpec((C, C), full),
            pl.BlockSpec((C, C), full),
            pl.BlockSpec((1, C), full),
        ],
        out_specs=pl.BlockSpec((PB, C), lambda i: (i, 0)),
        out_shape=jax.ShapeDtypeStruct((M, C), jnp.float32),
    )(acc, sl, sr, wt, wl, wr, b.reshape(1, C))


def _attn_body(par_ref, nd0_ref, sl_ref, sr_ref, wt_ref, wl_ref, wr_ref,
               b_ref, wq_ref, bq_ref, wk_ref, wv_ref, bv_ref, gate_ref,
               out_ref):
    acc = par_ref[0] + nd0_ref[0]
    r = jnp.dot(acc, wt_ref[...], preferred_element_type=jnp.float32)
    r += jnp.dot(sl_ref[0], wl_ref[...], preferred_element_type=jnp.float32)
    r += jnp.dot(sr_ref[0], wr_ref[...], preferred_element_type=jnp.float32)
    r += b_ref[...]
    x = jnp.where(r > 0, r, 0.01 * r)
    root = x[0:1, :]
    q = lax.dot_general(root, wq_ref[...], (((1,), (1,)), ((), ())),
                        preferred_element_type=jnp.float32) + bq_ref[...]
    kq = jnp.dot(q, wk_ref[...], preferred_element_type=jnp.float32)
    logits = lax.dot_general(x, kq, (((1,), (1,)), ((), ())),
                             preferred_element_type=jnp.float32)
    row = lax.broadcasted_iota(jnp.int32, (N, 1), 0)
    logits = jnp.where(row == 0, -1e30, logits)
    mx = jnp.max(logits)
    s = jnp.exp(logits - mx)
    s = jnp.where(row == 0, 0.0, s)
    z = jnp.sum(s)
    t = lax.dot_general(s, x, (((0,), (0,)), ((), ())),
                        preferred_element_type=jnp.float32)
    agg = lax.dot_general(t, wv_ref[...], (((1,), (1,)), ((), ())),
                          preferred_element_type=jnp.float32) / z
    agg = agg + bv_ref[...]
    g = jax.nn.sigmoid(gate_ref[0])
    out_ref[...] = (g * root + (1.0 - g) * agg).reshape(1, 1, C)


def _conv_attention(parent, node0, sl, sr, wt, wl, wr, b,
                    Wq, bq, Wk, Wv, bv, gate):
    full = lambda i: (0, 0)
    big = pl.BlockSpec((1, N, C), lambda i: (i, 0, 0))
    sq = pl.BlockSpec((C, C), full)
    row = pl.BlockSpec((1, C), full)
    return pl.pallas_call(
        _attn_body,
        grid=(B,),
        in_specs=[
            big, big, big, big,
            sq, sq, sq, row,
            sq, row, sq, sq, row,
            pl.BlockSpec(memory_space=pltpu.SMEM),
        ],
        out_specs=pl.BlockSpec((1, 1, C), lambda i: (i, 0, 0)),
        out_shape=jax.ShapeDtypeStruct((B, 1, C), jnp.float32),
    )(parent.reshape(B, N, C), node0.reshape(B, N, C),
      sl.reshape(B, N, C), sr.reshape(B, N, C),
      wt, wl, wr, b.reshape(1, C),
      Wq, bq.reshape(1, C), Wk, Wv, bv.reshape(1, C), gate).reshape(B, C)


def kernel(parent_node_embedding, children_index, w_t0, w_l0, w_r0, b0,
           w_t1, w_l1, w_r1, b1, Wq, bq, Wk, bk, Wv, bv, gate):
    parent = parent_node_embedding.reshape(M, C)
    ci = children_index.reshape(M, K)

    gidx, elr = _prep(ci)
    gidx_f = gidx.reshape(M * K)
    elr_f = elr.reshape(M * 2 * K)

    sl, sr = _sc_gather(parent, gidx_f, elr_f)
    node0 = _conv(parent, sl, sr, w_t0, w_l0, w_r0, b0)

    sl, sr = _sc_gather(node0, gidx_f, elr_f)
    return _conv_attention(parent, node0, sl, sr, w_t1, w_l1, w_r1, b1,
                           Wq, bq, Wk, Wv, bv, gate)


# speedup vs baseline: 1.1849x; 1.0475x over previous
import functools

import jax
import jax.numpy as jnp
from jax import lax
from jax.experimental import pallas as pl
from jax.experimental.pallas import tpu as pltpu
from jax.experimental.pallas import tpu_sc as plsc

B, N, K, C = 4, 8192, 8, 128
M = B * N
NC, NS, L = 2, 16, 16
NW = NC * NS
NPT = M // NW
CH = 32
NCHUNK = NPT // CH
PB = 2048


def _prep_body(ci_ref, gidx_ref, elr_ref):
    pid = pl.program_id(0)
    base = (pid * PB // N) * N
    ci = ci_ref[...]
    m = (ci != 0).astype(jnp.float32)
    num_sib = jnp.sum(m, axis=1, keepdims=True)
    is1 = num_sib == 1.0
    denom = jnp.where(is1, 1.0, num_sib - 1.0)
    kidx = lax.broadcasted_iota(jnp.int32, (PB, K), 1).astype(jnp.float32)
    er_full = jnp.where(is1, jnp.where(kidx == 0.0, 0.5, 0.0),
                        kidx * m / denom)
    elr_ref[...] = jnp.concatenate([m * (1.0 - er_full), m * er_full], axis=1)
    gidx_ref[...] = ci + base


def _prep(ci_flat):
    return pl.pallas_call(
        _prep_body,
        grid=(M // PB,),
        in_specs=[pl.BlockSpec((PB, K), lambda i: (i, 0))],
        out_specs=[pl.BlockSpec((PB, K), lambda i: (i, 0)),
                   pl.BlockSpec((PB, 2 * K), lambda i: (i, 0))],
        out_shape=[
            jax.ShapeDtypeStruct((M, K), jnp.int32),
            jax.ShapeDtypeStruct((M, 2 * K), jnp.float32),
        ],
    )(ci_flat)


_BCAST_DN = lax.GatherDimensionNumbers(
    offset_dims=(), collapsed_slice_dims=(0,), start_index_map=(0,))


def _lane_bcast(vec16, k):
    idx = jnp.full((16, 1), k, jnp.int32)
    return lax.gather(vec16, idx, _BCAST_DN, slice_sizes=(1,),
                      mode=lax.GatherScatterMode.PROMISE_IN_BOUNDS)


def _sc_body(table, gidx, elr, sl, sr,
             idx0, idx1, idx2, idx3, w0, w1, w2, w3, rows0, rows1,
             sl0, sl1, sr0, sr1,
             gsem0, gsem1, asem0, asem1, asem2, asem3):
    wid = lax.axis_index("s") * NC + lax.axis_index("c")
    nbase = wid * NPT
    idx = [idx0, idx1, idx2, idx3]
    wv = [w0, w1, w2, w3]
    rows = [rows0, rows1]
    stl = [sl0, sl1]
    str_ = [sr0, sr1]
    gsem = [gsem0, gsem1]
    asem = [asem0, asem1, asem2, asem3]

    def issue_a(q, b4):
        col = nbase + q * CH
        pltpu.async_copy(gidx.at[pl.ds(col * K, CH * K)], idx[b4], asem[b4])
        pltpu.async_copy(elr.at[pl.ds(col * 2 * K, CH * 2 * K)], wv[b4],
                         asem[b4])

    def wait_a(b4):
        pltpu.make_async_copy(gidx.at[pl.ds(0, CH * K)], idx[b4],
                              asem[b4]).wait()
        pltpu.make_async_copy(elr.at[pl.ds(0, CH * 2 * K)], wv[b4],
                              asem[b4]).wait()

    def issue_b(b4, rb):
        pltpu.async_copy(table.at[idx[b4]], rows[rb], gsem[rb])

    issue_a(0, 0)
    issue_a(1, 1)
    wait_a(0)
    issue_b(0, 0)

    def outer(qq, carry):
        for b in range(4):
            q = qq * 4 + b
            rb = b % 2

            @pl.when(q + 1 < NCHUNK)
            def _():
                wait_a((b + 1) % 4)
                issue_b((b + 1) % 4, (rb + 1) % 2)

            pltpu.make_async_copy(table.at[idx[b]], rows[rb],
                                  gsem[rb]).wait()

            rows_v, elr_v, stagl, stagr = rows[rb], wv[b], stl[rb], str_[rb]

            def node_body(i, c3):
                w16 = elr_v[pl.ds(i * (2 * K), 16)]
                accl = [None] * (C // 16)
                accr = [None] * (C // 16)
                for k in range(K):
                    bl = _lane_bcast(w16, k)
                    br = _lane_bcast(w16, K + k)
                    row = i * K + k
                    for v in range(C // 16):
                        r = rows_v[row, pl.ds(v * 16, 16)]
                        if k == 0:
                            accl[v] = bl * r
                            accr[v] = br * r
                        else:
                            accl[v] = accl[v] + bl * r
                            accr[v] = accr[v] + br * r
                for v in range(C // 16):
                    stagl[i, pl.ds(v * 16, 16)] = accl[v]
                    stagr[i, pl.ds(v * 16, 16)] = accr[v]
                return c3

            lax.fori_loop(0, CH, node_body, 0, unroll=4)

            col = nbase + q * CH
            pltpu.sync_copy(stagl, sl.at[pl.ds(col, CH), :])
            pltpu.sync_copy(stagr, sr.at[pl.ds(col, CH), :])

            @pl.when(q + 2 < NCHUNK)
            def _():
                issue_a(q + 2, (b + 2) % 4)
        return carry

    lax.fori_loop(0, NCHUNK // 4, outer, 0)


def _sc_gather(table, gidx_f, elr_f):
    mesh = plsc.VectorSubcoreMesh(core_axis_name="c", subcore_axis_name="s",
                                  num_cores=NC, num_subcores=NS)
    f = pl.kernel(
        _sc_body,
        out_type=[
            jax.ShapeDtypeStruct((M, C), jnp.float32),
            jax.ShapeDtypeStruct((M, C), jnp.float32),
        ],
        mesh=mesh,
        scratch_types=(
            [pltpu.VMEM((CH * K,), jnp.int32)] * 4
            + [pltpu.VMEM((CH * 2 * K,), jnp.float32)] * 4
            + [pltpu.VMEM((CH * K, C), jnp.float32)] * 2
            + [pltpu.VMEM((CH, C), jnp.float32)] * 4
            + [pltpu.SemaphoreType.DMA] * 6
        ),
        compiler_params=pltpu.CompilerParams(needs_layout_passes=False,
                                             disable_bounds_checks=True),
    )
    return f(table, gidx_f, elr_f)


def _conv0_body(acc_ref, sl_ref, sr_ref, wt_ref, wl_ref, wr_ref, b_ref,
                node_ref):
    x = acc_ref[...]
    r = jnp.dot(x, wt_ref[...], preferred_element_type=jnp.float32)
    r += jnp.dot(sl_ref[...], wl_ref[...], preferred_element_type=jnp.float32)
    r += jnp.dot(sr_ref[...], wr_ref[...], preferred_element_type=jnp.float32)
    r += b_ref[...]
    node_ref[...] = jnp.where(r > 0, r, 0.01 * r)


def _conv(acc, sl, sr, wt, wl, wr, b):
    full = lambda i: (0, 0)
    return pl.pallas_call(
        _conv0_body,
        grid=(M // PB,),
        in_specs=[
            pl.BlockSpec((PB, C), lambda i: (i, 0)),
            pl.BlockSpec((PB, C), lambda i: (i, 0)),
            pl.BlockSpec((PB, C), lambda i: (i, 0)),
            pl.BlockSpec((C, C), full),
            pl.BlockSpec((C, C), full),
            pl.BlockSpec((C, C), full),
            pl.BlockSpec((1, C), full),
        ],
        out_specs=pl.BlockSpec((PB, C), lambda i: (i, 0)),
        out_shape=jax.ShapeDtypeStruct((M, C), jnp.float32),
    )(acc, sl, sr, wt, wl, wr, b.reshape(1, C))


def _attn_body(par_ref, nd0_ref, sl_ref, sr_ref, wt_ref, wl_ref, wr_ref,
               b_ref, wq_ref, bq_ref, wk_ref, wv_ref, bv_ref, gate_ref,
               out_ref):
    acc = par_ref[0] + nd0_ref[0]
    r = jnp.dot(acc, wt_ref[...], preferred_element_type=jnp.float32)
    r += jnp.dot(sl_ref[0], wl_ref[...], preferred_element_type=jnp.float32)
    r += jnp.dot(sr_ref[0], wr_ref[...], preferred_element_type=jnp.float32)
    r += b_ref[...]
    x = jnp.where(r > 0, r, 0.01 * r)
    root = x[0:1, :]
    q = lax.dot_general(root, wq_ref[...], (((1,), (1,)), ((), ())),
                        preferred_element_type=jnp.float32) + bq_ref[...]
    kq = jnp.dot(q, wk_ref[...], preferred_element_type=jnp.float32)
    logits = lax.dot_general(x, kq, (((1,), (1,)), ((), ())),
                             preferred_element_type=jnp.float32)
    row = lax.broadcasted_iota(jnp.int32, (N, 1), 0)
    logits = jnp.where(row == 0, -1e30, logits)
    mx = jnp.max(logits)
    s = jnp.exp(logits - mx)
    s = jnp.where(row == 0, 0.0, s)
    z = jnp.sum(s)
    t = lax.dot_general(s, x, (((0,), (0,)), ((), ())),
                        preferred_element_type=jnp.float32)
    agg = lax.dot_general(t, wv_ref[...], (((1,), (1,)), ((), ())),
                          preferred_element_type=jnp.float32) / z
    agg = agg + bv_ref[...]
    g = jax.nn.sigmoid(gate_ref[0])
    out_ref[...] = (g * root + (1.0 - g) * agg).reshape(1, 1, C)


def _conv_attention(parent, node0, sl, sr, wt, wl, wr, b,
                    Wq, bq, Wk, Wv, bv, gate):
    full = lambda i: (0, 0)
    big = pl.BlockSpec((1, N, C), lambda i: (i, 0, 0))
    sq = pl.BlockSpec((C, C), full)
    row = pl.BlockSpec((1, C), full)
    return pl.pallas_call(
        _attn_body,
        grid=(B,),
        in_specs=[
            big, big, big, big,
            sq, sq, sq, row,
            sq, row, sq, sq, row,
            pl.BlockSpec(memory_space=pltpu.SMEM),
        ],
        out_specs=pl.BlockSpec((1, 1, C), lambda i: (i, 0, 0)),
        out_shape=jax.ShapeDtypeStruct((B, 1, C), jnp.float32),
    )(parent.reshape(B, N, C), node0.reshape(B, N, C),
      sl.reshape(B, N, C), sr.reshape(B, N, C),
      wt, wl, wr, b.reshape(1, C),
      Wq, bq.reshape(1, C), Wk, Wv, bv.reshape(1, C), gate).reshape(B, C)


def kernel(parent_node_embedding, children_index, w_t0, w_l0, w_r0, b0,
           w_t1, w_l1, w_r1, b1, Wq, bq, Wk, bk, Wv, bv, gate):
    parent = parent_node_embedding.reshape(M, C)
    ci = children_index.reshape(M, K)

    gidx, elr = _prep(ci)
    gidx_f = gidx.reshape(M * K)
    elr_f = elr.reshape(M * 2 * K)

    sl, sr = _sc_gather(parent, gidx_f, elr_f)
    node0 = _conv(parent, sl, sr, w_t0, w_l0, w_r0, b0)

    sl, sr = _sc_gather(node0, gidx_f, elr_f)
    return _conv_attention(parent, node0, sl, sr, w_t1, w_l1, w_r1, b1,
                           Wq, bq, Wk, Wv, bv, gate)
